# Initial kernel scaffold; baseline (speedup 1.0000x reference)
import jax
import jax.numpy as jnp
from jax.experimental import pallas as pl


def kernel(x, edge_index, W1, b1, W2, b2):
    raise NotImplementedError("write your pallas kernel here")


# speedup vs baseline: 12.0395x; 12.0395x over previous
import functools

import jax
import jax.numpy as jnp
from jax import lax
from jax.experimental import pallas as pl
from jax.experimental.pallas import tpu as pltpu
from jax.experimental.pallas import tpu_sc as plsc

_NC = 2
_NS = 16
_NW = _NC * _NS
_CHUNK = 128


def _sc_mesh():
    return plsc.VectorSubcoreMesh(
        core_axis_name="c", subcore_axis_name="s", num_cores=_NC, num_subcores=_NS
    )


def _sc_degree(dst3, zeros_hbm, npad, nch):
    rps = npad // _NS

    @functools.partial(
        pl.kernel,
        out_type=jax.ShapeDtypeStruct((_NC, npad), jnp.float32),
        mesh=_sc_mesh(),
        scratch_types=[
            pltpu.VMEM((nch, _CHUNK), jnp.int32),
            pltpu.VMEM((_CHUNK,), jnp.float32),
            pltpu.VMEM_SHARED((npad,), jnp.float32),
        ],
    )
    def k(dst_hbm, zeros_h, ones_h, out_hbm, didx, ones_v, acc):
        c = lax.axis_index("c")
        s = lax.axis_index("s")
        wid = s * _NC + c
        r0 = s * rps
        pltpu.sync_copy(zeros_h.at[pl.ds(r0, rps)], acc.at[pl.ds(r0, rps)])
        pltpu.sync_copy(ones_h, ones_v)
        pltpu.sync_copy(dst_hbm.at[wid], didx)
        plsc.subcore_barrier()

        def body(i, carry):
            pltpu.sync_copy(ones_v, acc.at[didx.at[i]], add=True)
            return carry

        lax.fori_loop(0, nch, body, 0)
        plsc.subcore_barrier()
        pltpu.sync_copy(acc.at[pl.ds(r0, rps)], out_hbm.at[c, pl.ds(r0, rps)])

    return k(dst3, zeros_hbm, jnp.ones((_CHUNK,), jnp.float32))


def _sc_edge_agg(h, src3, dst3, zeros_hbm, npad, nch):
    n, d = h.shape
    rps = npad // _NS

    @functools.partial(
        pl.kernel,
        out_type=jax.ShapeDtypeStruct((_NC, npad, d), jnp.float32),
        mesh=_sc_mesh(),
        scratch_types=[
            pltpu.VMEM((nch, _CHUNK), jnp.int32),
            pltpu.VMEM((nch, _CHUNK), jnp.int32),
            pltpu.VMEM((_CHUNK, d), jnp.float32),
            pltpu.VMEM_SHARED((npad, d), jnp.float32),
            pltpu.SemaphoreType.DMA,
        ],
    )
    def k(h_hbm, src_hbm, dst_hbm, zeros_h, out_hbm, sidx, didx, rows, acc, sem):
        c = lax.axis_index("c")
        s = lax.axis_index("s")
        wid = s * _NC + c
        r0 = s * rps
        pltpu.sync_copy(zeros_h, acc.at[pl.ds(r0, rps)])
        pltpu.sync_copy(src_hbm.at[wid], sidx)
        pltpu.sync_copy(dst_hbm.at[wid], didx)
        plsc.subcore_barrier()

        def body(i, carry):
            pltpu.async_copy(h_hbm.at[sidx.at[i]], rows, sem).wait()
            pltpu.sync_copy(rows, acc.at[didx.at[i]], add=True)
            return carry

        lax.fori_loop(0, nch, body, 0)
        plsc.subcore_barrier()
        pltpu.sync_copy(acc.at[pl.ds(r0, rps)], out_hbm.at[c, pl.ds(r0, rps)])

    return k(h, src3, dst3, zeros_hbm)


def _tc_layer1(x, w1, d0, d1, bm=1000):
    m, d = x.shape

    def body(x_ref, w_ref, d0_ref, d1_ref, o_ref):
        dis = lax.rsqrt(d0_ref[...] + d1_ref[...] + 1.0)
        o_ref[...] = (
            jnp.dot(x_ref[...], w_ref[...], preferred_element_type=jnp.float32) * dis
        )

    return pl.pallas_call(
        body,
        grid=(m // bm,),
        in_specs=[
            pl.BlockSpec((bm, d), lambda i: (i, 0)),
            pl.BlockSpec((d, d), lambda i: (0, 0)),
            pl.BlockSpec((bm, 1), lambda i: (i, 0)),
            pl.BlockSpec((bm, 1), lambda i: (i, 0)),
        ],
        out_specs=pl.BlockSpec((bm, d), lambda i: (i, 0)),
        out_shape=jax.ShapeDtypeStruct((m, d), jnp.float32),
    )(x, w1, d0, d1)


def _tc_layer2(p0, p1, h1p, d0, d1, b1, w2, bm=1000):
    m, d = h1p.shape

    def body(p0_ref, p1_ref, h_ref, d0_ref, d1_ref, b_ref, w_ref, o_ref):
        dis = lax.rsqrt(d0_ref[...] + d1_ref[...] + 1.0)
        z = dis * (p0_ref[...] + p1_ref[...] + h_ref[...]) + b_ref[...]
        z = jnp.maximum(z, 0.0)
        o_ref[...] = (
            jnp.dot(z, w_ref[...], preferred_element_type=jnp.float32) * dis
        )

    row = pl.BlockSpec((bm, d), lambda i: (i, 0))
    return pl.pallas_call(
        body,
        grid=(m // bm,),
        in_specs=[
            row,
            row,
            row,
            pl.BlockSpec((bm, 1), lambda i: (i, 0)),
            pl.BlockSpec((bm, 1), lambda i: (i, 0)),
            pl.BlockSpec((1, d), lambda i: (0, 0)),
            pl.BlockSpec((d, d), lambda i: (0, 0)),
        ],
        out_specs=row,
        out_shape=jax.ShapeDtypeStruct((m, d), jnp.float32),
    )(p0, p1, h1p, d0, d1, b1, w2)


def _tc_layer3(q0, q1, h2p, d0, d1, b2, bm=1000):
    m, d = h2p.shape

    def body(q0_ref, q1_ref, h_ref, d0_ref, d1_ref, b_ref, o_ref):
        dis = lax.rsqrt(d0_ref[...] + d1_ref[...] + 1.0)
        o_ref[...] = dis * (q0_ref[...] + q1_ref[...] + h_ref[...]) + b_ref[...]

    row = pl.BlockSpec((bm, d), lambda i: (i, 0))
    return pl.pallas_call(
        body,
        grid=(m // bm,),
        in_specs=[
            row,
            row,
            row,
            pl.BlockSpec((bm, 1), lambda i: (i, 0)),
            pl.BlockSpec((bm, 1), lambda i: (i, 0)),
            pl.BlockSpec((1, d), lambda i: (0, 0)),
        ],
        out_specs=row,
        out_shape=jax.ShapeDtypeStruct((m, d), jnp.float32),
    )(q0, q1, h2p, d0, d1, b2)


def kernel(x, edge_index, W1, b1, W2, b2):
    n, d = x.shape
    e = edge_index.shape[1]

    nch = -(-e // (_NW * _CHUNK))
    epw = nch * _CHUNK
    ep = epw * _NW
    npad = 256 * (-(-(n + 1) // 256))

    src = edge_index[0].astype(jnp.int32)
    dst = edge_index[1].astype(jnp.int32)
    pad = ep - e
    src3 = jnp.concatenate([src, jnp.zeros((pad,), jnp.int32)]).reshape(
        _NW, nch, _CHUNK
    )
    dst3 = jnp.concatenate(
        [dst, jnp.full((pad,), n, jnp.int32)]
    ).reshape(_NW, nch, _CHUNK)

    rps = npad // _NS
    zeros_1d = jnp.zeros((npad,), jnp.float32)
    zeros_d = jnp.zeros((rps, d), jnp.float32)

    degp = _sc_degree(dst3, zeros_1d, npad, nch)
    d0 = degp[0, :n].reshape(n, 1)
    d1 = degp[1, :n].reshape(n, 1)

    h1p = _tc_layer1(x, W1, d0, d1)

    p = _sc_edge_agg(h1p, src3, dst3, zeros_d, npad, nch)
    h2p = _tc_layer2(p[0, :n, :], p[1, :n, :], h1p, d0, d1, b1.reshape(1, d), W2)

    q = _sc_edge_agg(h2p, src3, dst3, zeros_d, npad, nch)
    return _tc_layer3(q[0, :n, :], q[1, :n, :], h2p, d0, d1, b2.reshape(1, d))
